# baseline (speedup 1.0000x reference)
import functools

import jax
import jax.numpy as jnp
from jax import lax
from jax.experimental import pallas as pl
from jax.experimental.pallas import tpu as pltpu
from jax.experimental.pallas import tpu_sc as plsc

B = 16384
S = 10
F = 32
NC = 2
NS = 16
NW = NC * NS
BPW = B // NW
CB = 16
NCHUNK = BPW // CB
CN = CB * S
DGR = 62500
UGR = 6250

_mesh = plsc.VectorSubcoreMesh(core_axis_name="c", subcore_axis_name="s")


@functools.partial(
    pl.kernel,
    mesh=_mesh,
    compiler_params=pltpu.CompilerParams(use_tc_tiling_on_sc=False,
                                         needs_layout_passes=False),
    out_type=jax.ShapeDtypeStruct((S, B), jnp.float32),
    scratch_types=[
        pltpu.VMEM((CN,), jnp.int32),
        pltpu.VMEM((CB,), jnp.int32),
        pltpu.VMEM((CN * F,), jnp.int32),
        pltpu.VMEM((CB * F,), jnp.int32),
        pltpu.VMEM((CN * F, 16), jnp.float32),
        pltpu.VMEM((CB * F, 16), jnp.float32),
        pltpu.VMEM((S, BPW), jnp.float32),
        pltpu.SemaphoreType.DMA,
    ],
)
def _score_kernel(items_hbm, user_hbm, docg_hbm, userg_hbm, out_hbm,
                  iidx_v, uidx_v, gdi_v, gui_v, dgr_v, ugr_v, sc_v, sem):
    wid = lax.axis_index("c") * NS + lax.axis_index("s")
    lane = lax.iota(jnp.int32, 16)

    def chunk_body(chunk, carry):
        bbase = wid * BPW + chunk * CB
        for s in range(S):
            pltpu.sync_copy(items_hbm.at[s, pl.ds(bbase, CB)],
                            iidx_v.at[pl.ds(s * CB, CB)])
        pltpu.sync_copy(user_hbm.at[pl.ds(bbase, CB)], uidx_v)

        def didx_body(v, c1):
            base = iidx_v[pl.ds(v * 16, 16)] >> 4

            def cbody(c, c2):
                gdi_v[pl.ds((v * F + c) * 16, 16)] = base + c * DGR
                return c2

            lax.fori_loop(0, F, cbody, 0)
            return c1

        lax.fori_loop(0, CN // 16, didx_body, 0)
        ubase = uidx_v[pl.ds(0, 16)] >> 4

        def ubody(c, c2):
            gui_v[pl.ds(c * 16, 16)] = ubase + c * UGR
            return c2

        lax.fori_loop(0, F, ubody, 0)

        cp_doc = pltpu.async_copy(docg_hbm.at[gdi_v], dgr_v, sem)
        cp_usr = pltpu.async_copy(userg_hbm.at[gui_v], ugr_v, sem)
        cp_doc.wait()
        cp_usr.wait()

        upos = uidx_v[pl.ds(0, 16)] & 15

        def score_body(v, c1):
            pos = iidx_v[pl.ds(v * 16, 16)] & 15

            def cbody(c, acc):
                drow = (v * F + c) * 16 + lane
                urow = c * 16 + lane
                dval = plsc.load_gather(dgr_v, [drow, pos])
                uval = plsc.load_gather(ugr_v, [urow, upos])
                return acc + dval * uval

            acc = lax.fori_loop(0, F, cbody, jnp.zeros((16,), jnp.float32))
            sc_v[v, pl.ds(chunk * CB, 16)] = acc
            return c1

        lax.fori_loop(0, S, score_body, 0)
        return carry

    lax.fori_loop(0, NCHUNK, chunk_body, 0)
    for s in range(S):
        pltpu.sync_copy(sc_v.at[s], out_hbm.at[s, pl.ds(wid * BPW, BPW)])


def kernel(item_ids, user_ids, docEmbed, userEmbed):
    items_t = item_ids.T.astype(jnp.int32)
    uids = user_ids.astype(jnp.int32)
    docg = docEmbed.T.reshape(F * DGR, 16)
    userg = userEmbed.T.reshape(F * UGR, 16)
    out_t = _score_kernel(items_t, uids, docg, userg)
    return out_t.T


# speedup vs baseline: 5.2855x; 5.2855x over previous
import functools

import jax
import jax.numpy as jnp
from jax import lax
from jax.experimental import pallas as pl
from jax.experimental.pallas import tpu as pltpu
from jax.experimental.pallas import tpu_sc as plsc

B = 16384
S = 10
F = 32
NC = 2
NS = 16
NW = NC * NS
BPW = B // NW
CB = 256
NCHUNK = BPW // CB
CN = CB * S
GB = 8
GN = GB * S
NVEC = GN // 16

_mesh = plsc.VectorSubcoreMesh(core_axis_name="c", subcore_axis_name="s")


def _hsum_all_lanes(p, lane):
    for sft in (8, 4, 2, 1):
        p = p + jnp.take_along_axis(p, jnp.bitwise_xor(lane, sft), axis=0)
    return p


@functools.partial(
    pl.kernel,
    mesh=_mesh,
    compiler_params=pltpu.CompilerParams(use_tc_tiling_on_sc=False,
                                         needs_layout_passes=False),
    out_type=jax.ShapeDtypeStruct((S, B), jnp.float32),
    scratch_types=[
        pltpu.VMEM((CN,), jnp.int32),
        pltpu.VMEM((CB,), jnp.int32),
        pltpu.VMEM((CN, F), jnp.float32),
        pltpu.VMEM((CB, F), jnp.float32),
        pltpu.VMEM((CN,), jnp.float32),
        pltpu.SemaphoreType.DMA,
    ],
)
def _score_kernel(items_hbm, user_hbm, doc_hbm, uemb_hbm, out_hbm,
                  iidx_v, uidx_v, doc_v, usr_v, sc_v, sem):
    wid = lax.axis_index("c") * NS + lax.axis_index("s")
    lane = lax.iota(jnp.int32, 16)

    def chunk_body(chunk, carry):
        bbase = wid * BPW + chunk * CB
        for s in range(S):
            pltpu.sync_copy(items_hbm.at[s, pl.ds(bbase, CB)],
                            iidx_v.at[pl.ds(s * CB, CB)])
        pltpu.sync_copy(user_hbm.at[pl.ds(bbase, CB)], uidx_v)
        cp_doc = pltpu.async_copy(doc_hbm.at[iidx_v], doc_v, sem)
        cp_usr = pltpu.async_copy(uemb_hbm.at[uidx_v], usr_v, sem)
        cp_doc.wait()
        cp_usr.wait()

        def block_body(g, bcarry):
            base_b = (g % (CB // 16)) * 16
            base_n = (g // (CB // 16)) * CB + base_b
            acc = jnp.zeros((16,), jnp.float32)
            for l in range(16):
                u0 = usr_v[base_b + l, pl.ds(0, 16)]
                u1 = usr_v[base_b + l, pl.ds(16, 16)]
                d0 = doc_v[base_n + l, pl.ds(0, 16)]
                d1 = doc_v[base_n + l, pl.ds(16, 16)]
                tot = _hsum_all_lanes(d0 * u0 + d1 * u1, lane)
                acc = jnp.where(lane == l, tot, acc)
            sc_v[pl.ds(base_n, 16)] = acc
            return bcarry

        lax.fori_loop(0, CN // 16, block_body, 0)
        for s in range(S):
            pltpu.sync_copy(sc_v.at[pl.ds(s * CB, CB)],
                            out_hbm.at[s, pl.ds(bbase, CB)])
        return carry

    lax.fori_loop(0, NCHUNK, chunk_body, 0)


def kernel(item_ids, user_ids, docEmbed, userEmbed):
    items_t = item_ids.T.astype(jnp.int32)
    uids = user_ids.astype(jnp.int32)
    out_t = _score_kernel(items_t, uids, docEmbed, userEmbed)
    return out_t.T
